# Initial kernel scaffold; baseline (speedup 1.0000x reference)
import jax
import jax.numpy as jnp
from jax.experimental import pallas as pl


def kernel(r, coords, nuc_embed, spin_embed, W_ee, W_ne, W_upd, b_upd, w_rbf_ee, w_rbf_ne):
    raise NotImplementedError("write your pallas kernel here")


# speedup vs baseline: 1.8145x; 1.8145x over previous
import functools

import jax
import jax.numpy as jnp
import numpy as np
from jax.experimental import pallas as pl
from jax.experimental.pallas import tpu as pltpu

_N_UP = 1024
_CUTOFF = 10.0
_N_RBF = 16
_MU = [float(m) for m in np.linspace(0.0, _CUTOFF, _N_RBF, dtype=np.float32)]
_INV_SIG2 = float(1.0 / np.float32((_CUTOFF / _N_RBF) ** 2))

_INTERPRET = False


def _sall_body(r_ref, rT_ref, wee_ref, s_ref, *, ti, tj, n_layers):
    i = pl.program_id(0)
    j = pl.program_id(1)
    d2 = jnp.zeros((ti, tj), jnp.float32)
    for c in range(3):
        dx = r_ref[:, c:c + 1] - rT_ref[c:c + 1, :]
        d2 = d2 + dx * dx
    d = jnp.sqrt(d2 + 1e-12)
    row = jax.lax.broadcasted_iota(jnp.int32, (ti, tj), 0) + i * ti
    col = jax.lax.broadcasted_iota(jnp.int32, (ti, tj), 1) + j * tj
    env = jnp.where((d < _CUTOFF) & (row != col),
                    (1.0 - d * (1.0 / _CUTOFF)) ** 2, 0.0)
    accs = [jnp.zeros((ti, tj), jnp.float32) for _ in range(n_layers)]
    for k in range(_N_RBF):
        e = jnp.exp((d - _MU[k]) ** 2 * (-_INV_SIG2))
        for l in range(n_layers):
            accs[l] = accs[l] + wee_ref[l, k] * e
    for l in range(n_layers):
        s_ref[l] = env * accs[l]


def _layer_body(s_ref, h_ref, r_ref, coordsT_ref, hnuc_ref, wee_ref, wne_ref,
                wu_ref, b_ref, wrbfne_ref, hout_ref, *, ti, n_nuc, dim):
    i = pl.program_id(0)
    S = s_ref[0]
    h = h_ref[...]
    A = jnp.dot(S, h, preferred_element_type=jnp.float32)
    d2 = jnp.zeros((ti, n_nuc), jnp.float32)
    for c in range(3):
        dx = r_ref[:, c:c + 1] - coordsT_ref[c:c + 1, :]
        d2 = d2 + dx * dx
    dne = jnp.sqrt(d2 + 1e-12)
    envne = jnp.where(dne < _CUTOFF, (1.0 - dne * (1.0 / _CUTOFF)) ** 2, 0.0)
    acc = jnp.zeros((ti, n_nuc), jnp.float32)
    for k in range(_N_RBF):
        acc = acc + wrbfne_ref[k] * jnp.exp((dne - _MU[k]) ** 2 * (-_INV_SIG2))
    Sne = envne * acc
    B = jnp.dot(Sne, hnuc_ref[...], preferred_element_type=jnp.float32)
    msg = (jnp.dot(A, wee_ref[...], preferred_element_type=jnp.float32)
           + jnp.dot(B, wne_ref[...], preferred_element_type=jnp.float32))
    hi = h_ref[pl.ds(i * ti, ti), :]
    pre = (jnp.dot(hi, wu_ref[:dim, :], preferred_element_type=jnp.float32)
           + jnp.dot(msg, wu_ref[dim:, :], preferred_element_type=jnp.float32)
           + b_ref[0, :])
    hout_ref[...] = hi + jnp.tanh(pre)


def kernel(r, coords, nuc_embed, spin_embed, W_ee, W_ne, W_upd, b_upd,
           w_rbf_ee, w_rbf_ne):
    n = r.shape[0]
    n_nuc = coords.shape[0]
    dim = nuc_embed.shape[1]
    n_layers = W_ee.shape[0]
    rT = r.T
    coordsT = coords.T

    ti = 256
    tj = 256
    s_all = pl.pallas_call(
        functools.partial(_sall_body, ti=ti, tj=tj, n_layers=n_layers),
        grid=(n // ti, n // tj),
        in_specs=[
            pl.BlockSpec((ti, 3), lambda i, j: (i, 0)),
            pl.BlockSpec((3, tj), lambda i, j: (0, j)),
            pl.BlockSpec(memory_space=pltpu.SMEM),
        ],
        out_specs=pl.BlockSpec((n_layers, ti, tj), lambda i, j: (0, i, j)),
        out_shape=jax.ShapeDtypeStruct((n_layers, n, n), jnp.float32),
        interpret=_INTERPRET,
    )(r, rT, w_rbf_ee)

    spin_idx = jnp.concatenate([
        jnp.zeros((_N_UP,), jnp.int32),
        jnp.ones((n - _N_UP,), jnp.int32),
    ])
    h = jnp.take(spin_embed, spin_idx, axis=0)

    tl = 256
    layer_call = pl.pallas_call(
        functools.partial(_layer_body, ti=tl, n_nuc=n_nuc, dim=dim),
        grid=(n // tl,),
        in_specs=[
            pl.BlockSpec((1, tl, n), lambda i: (0, i, 0)),
            pl.BlockSpec((n, dim), lambda i: (0, 0)),
            pl.BlockSpec((tl, 3), lambda i: (i, 0)),
            pl.BlockSpec((3, n_nuc), lambda i: (0, 0)),
            pl.BlockSpec((n_nuc, dim), lambda i: (0, 0)),
            pl.BlockSpec((dim, dim), lambda i: (0, 0)),
            pl.BlockSpec((dim, dim), lambda i: (0, 0)),
            pl.BlockSpec((2 * dim, dim), lambda i: (0, 0)),
            pl.BlockSpec((1, dim), lambda i: (0, 0)),
            pl.BlockSpec(memory_space=pltpu.SMEM),
        ],
        out_specs=pl.BlockSpec((tl, dim), lambda i: (i, 0)),
        out_shape=jax.ShapeDtypeStruct((n, dim), jnp.float32),
        interpret=_INTERPRET,
    )
    for l in range(n_layers):
        h = layer_call(s_all[l:l + 1], h, r, coordsT, nuc_embed,
                       W_ee[l], W_ne[l], W_upd[l], b_upd[l][None, :],
                       w_rbf_ne[l])
    return h
